# Initial kernel scaffold; baseline (speedup 1.0000x reference)
import jax
import jax.numpy as jnp
from jax.experimental import pallas as pl


def kernel(x, edge_index, edge_attr, params):
    raise NotImplementedError("write your pallas kernel here")


# speedup vs baseline: 3.0748x; 3.0748x over previous
import functools

import jax
import jax.numpy as jnp
from jax import lax
from jax.experimental import pallas as pl
from jax.experimental.pallas import tpu as pltpu
from jax.experimental.pallas import tpu_sc as plsc

N, E, F_IN, E_DIM, H = 10000, 320000, 128, 16, 100
HP = 128
DP = 128
LANES = 16
NC, NS = 2, 16
NW = NC * NS
EW = E // NW
CHUNK_A = 80
NCHUNK_A = EW // CHUNK_A
CHUNK = 400
NCHUNK = EW // CHUNK
NT = 624
BE = 2000


def _pad2(w, r, c):
    out = jnp.zeros((r, c), jnp.float32)
    return out.at[: w.shape[0], : w.shape[1]].set(w)


def _pad1(b, c):
    out = jnp.zeros((1, c), jnp.float32)
    return out.at[0, : b.shape[0]].set(b)


def _tc_node_init(x, w, b):
    def f(x_ref, w_ref, b_ref, o_ref):
        o_ref[...] = (
            jnp.dot(x_ref[...], w_ref[...], preferred_element_type=jnp.float32)
            + b_ref[...]
        )

    return pl.pallas_call(
        f, out_shape=jax.ShapeDtypeStruct((N, HP), jnp.float32)
    )(x, w, b)


def _tc_edge_init(edge_attr, w, b):
    def f(a_ref, w_ref, b_ref, o_ref):
        o_ref[...] = (
            jnp.dot(a_ref[...], w_ref[...], preferred_element_type=jnp.float32)
            + b_ref[...]
        )

    return pl.pallas_call(
        f,
        grid=(E // BE,),
        in_specs=[
            pl.BlockSpec((BE, E_DIM), lambda i: (i, 0)),
            pl.BlockSpec((E_DIM, HP), lambda i: (0, 0)),
            pl.BlockSpec((1, HP), lambda i: (0, 0)),
        ],
        out_specs=pl.BlockSpec((BE, HP), lambda i: (i, 0)),
        out_shape=jax.ShapeDtypeStruct((E, HP), jnp.float32),
        compiler_params=pltpu.CompilerParams(dimension_semantics=("parallel",)),
    )(edge_attr, w, b)


def _tc_node_update(h, agg, w1, b1, w2, b2, bg, bb, wa, wb):
    def f(h_ref, agg_ref, w1_ref, b1_ref, w2_ref, b2_ref, bg_ref, bb_ref,
          wa_ref, wb_ref, hn_ref, a_ref, b_ref):
        hv = h_ref[...]
        av = agg_ref[...]
        z = hv + av[:N] + av[N:]
        z = jnp.maximum(
            jnp.dot(z, w1_ref[...], preferred_element_type=jnp.float32)
            + b1_ref[...], 0.0)
        z = (jnp.dot(z, w2_ref[...], preferred_element_type=jnp.float32)
             + b2_ref[...])
        m = jnp.mean(z, axis=0, keepdims=True)
        zc = z - m
        v = jnp.mean(zc * zc, axis=0, keepdims=True)
        zbn = bg_ref[...] * zc * lax.rsqrt(v + 1e-5) + bb_ref[...]
        hn = (hv + jnp.maximum(zbn, 0.0)) * 0.5
        hn_ref[...] = hn
        a_ref[...] = jnp.dot(hn, wa_ref[...], preferred_element_type=jnp.float32)
        b_ref[...] = jnp.dot(hn, wb_ref[...], preferred_element_type=jnp.float32)

    return pl.pallas_call(
        f,
        out_shape=(
            jax.ShapeDtypeStruct((N, HP), jnp.float32),
            jax.ShapeDtypeStruct((N, HP), jnp.float32),
            jax.ShapeDtypeStruct((N, HP), jnp.float32),
        ),
    )(h, agg, w1, b1, w2, b2, bg, bb, wa, wb)


def _tc_edge_update(g, ea, wc, b1, w2, b2):
    def f(g_ref, ea_ref, wc_ref, b1_ref, w2_ref, b2_ref, o_ref):
        eav = ea_ref[...]
        t = jnp.maximum(
            g_ref[...]
            + jnp.dot(eav, wc_ref[...], preferred_element_type=jnp.float32)
            + b1_ref[...], 0.0)
        o_ref[...] = eav + (
            jnp.dot(t, w2_ref[...], preferred_element_type=jnp.float32)
            + b2_ref[...]) * 0.5

    return pl.pallas_call(
        f,
        grid=(E // BE,),
        in_specs=[
            pl.BlockSpec((BE, HP), lambda i: (i, 0)),
            pl.BlockSpec((BE, HP), lambda i: (i, 0)),
            pl.BlockSpec((HP, HP), lambda i: (0, 0)),
            pl.BlockSpec((1, HP), lambda i: (0, 0)),
            pl.BlockSpec((HP, HP), lambda i: (0, 0)),
            pl.BlockSpec((1, HP), lambda i: (0, 0)),
        ],
        out_specs=pl.BlockSpec((BE, HP), lambda i: (i, 0)),
        out_shape=jax.ShapeDtypeStruct((E, HP), jnp.float32),
        compiler_params=pltpu.CompilerParams(dimension_semantics=("parallel",)),
    )(g, ea, wc, b1, w2, b2)


def _tc_pq(h, wa, wb):
    def f(h_ref, wa_ref, wb_ref, p_ref, q_ref):
        hr = jnp.maximum(h_ref[...], 0.0)
        p_ref[...] = jnp.dot(hr, wa_ref[...], preferred_element_type=jnp.float32)
        q_ref[...] = jnp.dot(hr, wb_ref[...], preferred_element_type=jnp.float32)

    return pl.pallas_call(
        f,
        out_shape=(
            jax.ShapeDtypeStruct((N, DP), jnp.float32),
            jax.ShapeDtypeStruct((N, DP), jnp.float32),
        ),
    )(h, wa, wb)


def _tc_final(g, ea, wc, b1, w2, b2, w3, b3):
    def f(g_ref, ea_ref, wc_ref, b1_ref, w2_ref, b2_ref, w3_ref, b3_ref, o_ref):
        o1 = jnp.maximum(
            g_ref[...]
            + jnp.dot(ea_ref[...], wc_ref[...], preferred_element_type=jnp.float32)
            + b1_ref[...], 0.0)
        o2 = jnp.maximum(
            jnp.dot(o1, w2_ref[...], preferred_element_type=jnp.float32)
            + b2_ref[...], 0.0)
        o_ref[...] = (
            jnp.dot(o2, w3_ref[...], preferred_element_type=jnp.float32)
            + b3_ref[...])

    return pl.pallas_call(
        f,
        grid=(E // BE,),
        in_specs=[
            pl.BlockSpec((BE, DP), lambda i: (i, 0)),
            pl.BlockSpec((BE, HP), lambda i: (i, 0)),
            pl.BlockSpec((HP, DP), lambda i: (0, 0)),
            pl.BlockSpec((1, DP), lambda i: (0, 0)),
            pl.BlockSpec((DP, 32), lambda i: (0, 0)),
            pl.BlockSpec((1, 32), lambda i: (0, 0)),
            pl.BlockSpec((32, 8), lambda i: (0, 0)),
            pl.BlockSpec((1, 8), lambda i: (0, 0)),
        ],
        out_specs=pl.BlockSpec((BE, 8), lambda i: (i, 0)),
        out_shape=jax.ShapeDtypeStruct((E, 8), jnp.float32),
        compiler_params=pltpu.CompilerParams(dimension_semantics=("parallel",)),
    )(g, ea, wc, b1, w2, b2, w3, b3)


def _sc_mesh():
    return plsc.VectorSubcoreMesh(core_axis_name="c", subcore_axis_name="s")


def _sc_msg_agg(h, ea, src, dst, zeros_n):
    @functools.partial(
        pl.kernel,
        mesh=_sc_mesh(),
        out_type=jax.ShapeDtypeStruct((2 * N, HP), jnp.float32),
        scratch_types=[
            pltpu.VMEM((CHUNK_A,), jnp.int32),
            pltpu.VMEM((CHUNK_A,), jnp.int32),
            pltpu.VMEM((CHUNK_A, HP), jnp.float32),
            pltpu.VMEM((CHUNK_A, HP), jnp.float32),
            pltpu.VMEM_SHARED((N, HP), jnp.float32),
            pltpu.SemaphoreType.DMA,
        ],
    )
    def k(h_hbm, ea_hbm, src_hbm, dst_hbm, z_hbm, agg_hbm,
          idx_s, idx_d, rows, eab, shared, sem):
        c = lax.axis_index("c")
        s = lax.axis_index("s")
        wid = s * NC + c
        pltpu.sync_copy(z_hbm.at[pl.ds(s * NT, NT)], shared.at[pl.ds(s * NT, NT)])

        @pl.when(s == NS - 1)
        def _zero_tail():
            pltpu.sync_copy(z_hbm.at[pl.ds(NS * NT, N - NS * NT)],
                            shared.at[pl.ds(NS * NT, N - NS * NT)])

        plsc.subcore_barrier()

        def step(j, carry):
            base = wid * EW + j * CHUNK_A
            pltpu.sync_copy(src_hbm.at[pl.ds(base, CHUNK_A)], idx_s)
            pltpu.sync_copy(dst_hbm.at[pl.ds(base, CHUNK_A)], idx_d)
            pltpu.async_copy(h_hbm.at[idx_s], rows, sem).wait()
            pltpu.sync_copy(ea_hbm.at[pl.ds(base, CHUNK_A)], eab)

            def inner(e, carry2):
                for q in range(HP // LANES):
                    sl = pl.ds(q * LANES, LANES)
                    rows[e, sl] = jnp.maximum(rows[e, sl] + eab[e, sl], 0.0)
                return carry2

            lax.fori_loop(0, CHUNK_A, inner, 0)
            pltpu.sync_copy(rows, shared.at[idx_d], add=True)
            return carry

        lax.fori_loop(0, NCHUNK_A, step, 0)
        plsc.subcore_barrier()
        pltpu.sync_copy(shared.at[pl.ds(s * NT, NT)],
                        agg_hbm.at[pl.ds(c * N + s * NT, NT)])

        @pl.when(s == NS - 1)
        def _out_tail():
            pltpu.sync_copy(shared.at[pl.ds(NS * NT, N - NS * NT)],
                            agg_hbm.at[pl.ds(c * N + NS * NT, N - NS * NT)])

    return k(h, ea, src, dst, zeros_n)


def _sc_gather2(a, b, src, dst, d):
    @functools.partial(
        pl.kernel,
        mesh=_sc_mesh(),
        out_type=jax.ShapeDtypeStruct((E, d), jnp.float32),
        scratch_types=[
            pltpu.VMEM((CHUNK,), jnp.int32),
            pltpu.VMEM((CHUNK,), jnp.int32),
            pltpu.VMEM((CHUNK, d), jnp.float32),
            pltpu.VMEM((CHUNK, d), jnp.float32),
            pltpu.SemaphoreType.DMA,
            pltpu.SemaphoreType.DMA,
        ],
    )
    def k(a_hbm, b_hbm, src_hbm, dst_hbm, g_hbm,
          idx_s, idx_d, bufa, bufb, sem1, sem2):
        c = lax.axis_index("c")
        s = lax.axis_index("s")
        wid = s * NC + c

        def step(j, carry):
            base = wid * EW + j * CHUNK
            pltpu.sync_copy(src_hbm.at[pl.ds(base, CHUNK)], idx_s)
            pltpu.sync_copy(dst_hbm.at[pl.ds(base, CHUNK)], idx_d)
            cp1 = pltpu.async_copy(a_hbm.at[idx_s], bufa, sem1)
            cp2 = pltpu.async_copy(b_hbm.at[idx_d], bufb, sem2)
            cp1.wait()
            cp2.wait()

            def inner(e, carry2):
                for q in range(d // LANES):
                    sl = pl.ds(q * LANES, LANES)
                    bufa[e, sl] = bufa[e, sl] + bufb[e, sl]
                return carry2

            lax.fori_loop(0, CHUNK, inner, 0)
            pltpu.sync_copy(bufa, g_hbm.at[pl.ds(base, CHUNK)])
            return carry

        lax.fori_loop(0, NCHUNK, step, 0)

    return k(a, b, src, dst)


def kernel(x, edge_index, edge_attr, params):
    p = params
    src = edge_index[0]
    dst = edge_index[1]
    zeros_n = jnp.zeros((N, HP), jnp.float32)

    h = _tc_node_init(x, _pad2(p['node_W'], F_IN, HP), _pad1(p['node_b'], HP))
    ea = _tc_edge_init(edge_attr, _pad2(p['edge_W'], E_DIM, HP),
                       _pad1(p['edge_b'], HP))

    for i in range(2):
        agg = _sc_msg_agg(h, ea, src, dst, zeros_n)
        w1 = p['emlp%d_W1' % i]
        h, a, b = _tc_node_update(
            h, agg,
            _pad2(p['conv%d_W1' % i], HP, HP), _pad1(p['conv%d_b1' % i], HP),
            _pad2(p['conv%d_W2' % i], HP, HP), _pad1(p['conv%d_b2' % i], HP),
            _pad1(p['bn%d_g' % i], HP), _pad1(p['bn%d_b' % i], HP),
            _pad2(w1[:H], HP, HP), _pad2(w1[H:2 * H], HP, HP))
        g = _sc_gather2(a, b, src, dst, HP)
        ea = _tc_edge_update(
            g, ea,
            _pad2(w1[2 * H:], HP, HP), _pad1(p['emlp%d_b1' % i], HP),
            _pad2(p['emlp%d_W2' % i], HP, HP), _pad1(p['emlp%d_b2' % i], HP))

    w1 = p['mlp_W1']
    pn, qn = _tc_pq(h, _pad2(w1[:H], HP, DP), _pad2(w1[H:2 * H], HP, DP))
    gf = _sc_gather2(pn, qn, src, dst, DP)
    out = _tc_final(
        gf, ea,
        _pad2(w1[2 * H:], HP, DP), _pad1(p['mlp_b1'], DP),
        _pad2(p['mlp_W2'], DP, 32), _pad1(p['mlp_b2'], 32),
        _pad2(p['mlp_W3'], 32, 8), _pad1(p['mlp_b3'], 8))
    return out[:, :2]


# speedup vs baseline: 4.2011x; 1.3663x over previous
import functools

import jax
import jax.numpy as jnp
from jax import lax
from jax.experimental import pallas as pl
from jax.experimental.pallas import tpu as pltpu
from jax.experimental.pallas import tpu_sc as plsc

N, E, F_IN, E_DIM, H = 10000, 320000, 128, 16, 100
HP = 128
DP = 128
LANES = 16
NC, NS = 2, 16
NW = NC * NS
EW = E // NW
CHUNK_A = 80
NCHUNK_A = EW // CHUNK_A
CG = 80
NCG = EW // CG
NT = 624
BE = 2000


def _pad2(w, r, c):
    out = jnp.zeros((r, c), jnp.float32)
    return out.at[: w.shape[0], : w.shape[1]].set(w)


def _pad1(b, c):
    out = jnp.zeros((1, c), jnp.float32)
    return out.at[0, : b.shape[0]].set(b)


def _tc_node_init(x, w, b):
    def f(x_ref, w_ref, b_ref, o_ref):
        o_ref[...] = (
            jnp.dot(x_ref[...], w_ref[...], preferred_element_type=jnp.float32)
            + b_ref[...]
        )

    return pl.pallas_call(
        f, out_shape=jax.ShapeDtypeStruct((N, HP), jnp.float32)
    )(x, w, b)


def _tc_edge_init(edge_attr, w, b):
    def f(a_ref, w_ref, b_ref, o_ref):
        o_ref[...] = (
            jnp.dot(a_ref[...], w_ref[...], preferred_element_type=jnp.float32)
            + b_ref[...]
        )

    return pl.pallas_call(
        f,
        grid=(E // BE,),
        in_specs=[
            pl.BlockSpec((BE, E_DIM), lambda i: (i, 0)),
            pl.BlockSpec((E_DIM, HP), lambda i: (0, 0)),
            pl.BlockSpec((1, HP), lambda i: (0, 0)),
        ],
        out_specs=pl.BlockSpec((BE, HP), lambda i: (i, 0)),
        out_shape=jax.ShapeDtypeStruct((E, HP), jnp.float32),
        compiler_params=pltpu.CompilerParams(dimension_semantics=("parallel",)),
    )(edge_attr, w, b)


def _tc_node_update(h, agg, w1, b1, w2, b2, bg, bb, wa, wb):
    def f(h_ref, agg_ref, w1_ref, b1_ref, w2_ref, b2_ref, bg_ref, bb_ref,
          wa_ref, wb_ref, hn_ref, a_ref, b_ref):
        hv = h_ref[...]
        av = agg_ref[...]
        z = hv + av[:N] + av[N:]
        z = jnp.maximum(
            jnp.dot(z, w1_ref[...], preferred_element_type=jnp.float32)
            + b1_ref[...], 0.0)
        z = (jnp.dot(z, w2_ref[...], preferred_element_type=jnp.float32)
             + b2_ref[...])
        m = jnp.mean(z, axis=0, keepdims=True)
        zc = z - m
        v = jnp.mean(zc * zc, axis=0, keepdims=True)
        zbn = bg_ref[...] * zc * lax.rsqrt(v + 1e-5) + bb_ref[...]
        hn = (hv + jnp.maximum(zbn, 0.0)) * 0.5
        hn_ref[...] = hn
        a_ref[...] = jnp.dot(hn, wa_ref[...], preferred_element_type=jnp.float32)
        b_ref[...] = jnp.dot(hn, wb_ref[...], preferred_element_type=jnp.float32)

    return pl.pallas_call(
        f,
        out_shape=(
            jax.ShapeDtypeStruct((N, HP), jnp.float32),
            jax.ShapeDtypeStruct((N, HP), jnp.float32),
            jax.ShapeDtypeStruct((N, HP), jnp.float32),
        ),
    )(h, agg, w1, b1, w2, b2, bg, bb, wa, wb)


def _tc_edge_update(g, ea, wc, b1, w2, b2):
    def f(g_ref, ea_ref, wc_ref, b1_ref, w2_ref, b2_ref, o_ref):
        eav = ea_ref[...]
        t = jnp.maximum(
            g_ref[...]
            + jnp.dot(eav, wc_ref[...], preferred_element_type=jnp.float32)
            + b1_ref[...], 0.0)
        o_ref[...] = eav + (
            jnp.dot(t, w2_ref[...], preferred_element_type=jnp.float32)
            + b2_ref[...]) * 0.5

    return pl.pallas_call(
        f,
        grid=(E // BE,),
        in_specs=[
            pl.BlockSpec((BE, HP), lambda i: (i, 0)),
            pl.BlockSpec((BE, HP), lambda i: (i, 0)),
            pl.BlockSpec((HP, HP), lambda i: (0, 0)),
            pl.BlockSpec((1, HP), lambda i: (0, 0)),
            pl.BlockSpec((HP, HP), lambda i: (0, 0)),
            pl.BlockSpec((1, HP), lambda i: (0, 0)),
        ],
        out_specs=pl.BlockSpec((BE, HP), lambda i: (i, 0)),
        out_shape=jax.ShapeDtypeStruct((E, HP), jnp.float32),
        compiler_params=pltpu.CompilerParams(dimension_semantics=("parallel",)),
    )(g, ea, wc, b1, w2, b2)


def _tc_pq(h, wa, wb):
    def f(h_ref, wa_ref, wb_ref, p_ref, q_ref):
        hr = jnp.maximum(h_ref[...], 0.0)
        p_ref[...] = jnp.dot(hr, wa_ref[...], preferred_element_type=jnp.float32)
        q_ref[...] = jnp.dot(hr, wb_ref[...], preferred_element_type=jnp.float32)

    return pl.pallas_call(
        f,
        out_shape=(
            jax.ShapeDtypeStruct((N, DP), jnp.float32),
            jax.ShapeDtypeStruct((N, DP), jnp.float32),
        ),
    )(h, wa, wb)


def _tc_final(g, ea, wc, b1, w2, b2, w3, b3):
    def f(g_ref, ea_ref, wc_ref, b1_ref, w2_ref, b2_ref, w3_ref, b3_ref, o_ref):
        o1 = jnp.maximum(
            g_ref[...]
            + jnp.dot(ea_ref[...], wc_ref[...], preferred_element_type=jnp.float32)
            + b1_ref[...], 0.0)
        o2 = jnp.maximum(
            jnp.dot(o1, w2_ref[...], preferred_element_type=jnp.float32)
            + b2_ref[...], 0.0)
        o_ref[...] = (
            jnp.dot(o2, w3_ref[...], preferred_element_type=jnp.float32)
            + b3_ref[...])

    return pl.pallas_call(
        f,
        grid=(E // BE,),
        in_specs=[
            pl.BlockSpec((BE, DP), lambda i: (i, 0)),
            pl.BlockSpec((BE, HP), lambda i: (i, 0)),
            pl.BlockSpec((HP, DP), lambda i: (0, 0)),
            pl.BlockSpec((1, DP), lambda i: (0, 0)),
            pl.BlockSpec((DP, 32), lambda i: (0, 0)),
            pl.BlockSpec((1, 32), lambda i: (0, 0)),
            pl.BlockSpec((32, 8), lambda i: (0, 0)),
            pl.BlockSpec((1, 8), lambda i: (0, 0)),
        ],
        out_specs=pl.BlockSpec((BE, 8), lambda i: (i, 0)),
        out_shape=jax.ShapeDtypeStruct((E, 8), jnp.float32),
        compiler_params=pltpu.CompilerParams(dimension_semantics=("parallel",)),
    )(g, ea, wc, b1, w2, b2, w3, b3)


def _sc_mesh():
    return plsc.VectorSubcoreMesh(core_axis_name="c", subcore_axis_name="s")


def _sc_msg_agg(h, ea, src, dst, zeros_n):
    @functools.partial(
        pl.kernel,
        mesh=_sc_mesh(),
        out_type=jax.ShapeDtypeStruct((2 * N, HP), jnp.float32),
        scratch_types=[
            pltpu.VMEM((CHUNK_A,), jnp.int32),
            pltpu.VMEM((CHUNK_A,), jnp.int32),
            pltpu.VMEM((CHUNK_A,), jnp.int32),
            pltpu.VMEM((CHUNK_A,), jnp.int32),
            pltpu.VMEM((CHUNK_A, HP), jnp.float32),
            pltpu.VMEM((CHUNK_A, HP), jnp.float32),
            pltpu.VMEM((CHUNK_A, HP), jnp.float32),
            pltpu.VMEM((CHUNK_A, HP), jnp.float32),
            pltpu.VMEM_SHARED((N, HP), jnp.float32),
            pltpu.SemaphoreType.DMA,
            pltpu.SemaphoreType.DMA,
            pltpu.SemaphoreType.DMA,
            pltpu.SemaphoreType.DMA,
        ],
    )
    def k(h_hbm, ea_hbm, src_hbm, dst_hbm, z_hbm, agg_hbm,
          ids0, idd0, ids1, idd1, rows0, eab0, rows1, eab1, shared,
          sr0, se0, sr1, se1):
        c = lax.axis_index("c")
        s = lax.axis_index("s")
        wid = s * NC + c
        slots = ((ids0, idd0, rows0, eab0, sr0, se0),
                 (ids1, idd1, rows1, eab1, sr1, se1))

        pltpu.sync_copy(z_hbm.at[pl.ds(s * NT, NT)], shared.at[pl.ds(s * NT, NT)])

        @pl.when(s == NS - 1)
        def _zero_tail():
            pltpu.sync_copy(z_hbm.at[pl.ds(NS * NT, N - NS * NT)],
                            shared.at[pl.ds(NS * NT, N - NS * NT)])

        plsc.subcore_barrier()

        def issue(j, t):
            ids, idd, rows, eab, sr, se = slots[t]
            base = wid * EW + j * CHUNK_A
            pltpu.sync_copy(src_hbm.at[pl.ds(base, CHUNK_A)], ids)
            pltpu.sync_copy(dst_hbm.at[pl.ds(base, CHUNK_A)], idd)
            pltpu.async_copy(h_hbm.at[ids], rows, sr)
            pltpu.async_copy(ea_hbm.at[pl.ds(base, CHUNK_A)], eab, se)

        def wait_g(t):
            _, _, rows, eab, sr, se = slots[t]
            pltpu.make_async_copy(h_hbm.at[pl.ds(0, CHUNK_A)], rows, sr).wait()
            pltpu.make_async_copy(h_hbm.at[pl.ds(0, CHUNK_A)], eab, se).wait()

        def compute(t):
            _, _, rows, eab, _, _ = slots[t]

            @plsc.parallel_loop(0, CHUNK_A, unroll=4)
            def _(e):
                for q in range(HP // LANES):
                    sl = pl.ds(q * LANES, LANES)
                    rows[e, sl] = jnp.maximum(rows[e, sl] + eab[e, sl], 0.0)

        def scat(t):
            _, idd, rows, _, _, _ = slots[t]
            pltpu.sync_copy(rows, shared.at[idd], add=True)

        issue(0, 0)
        issue(1, 1)

        @pl.loop(0, (NCHUNK_A - 1) // 2)
        def _pairs(k2):
            g = k2 * 2
            wait_g(0)
            compute(0)
            scat(0)
            issue(g + 2, 0)
            wait_g(1)
            compute(1)
            scat(1)

            @pl.when(g + 3 < NCHUNK_A)
            def _():
                issue(g + 3, 1)

        wait_g(0)
        compute(0)
        scat(0)

        plsc.subcore_barrier()
        pltpu.sync_copy(shared.at[pl.ds(s * NT, NT)],
                        agg_hbm.at[pl.ds(c * N + s * NT, NT)])

        @pl.when(s == NS - 1)
        def _out_tail():
            pltpu.sync_copy(shared.at[pl.ds(NS * NT, N - NS * NT)],
                            agg_hbm.at[pl.ds(c * N + NS * NT, N - NS * NT)])

    return k(h, ea, src, dst, zeros_n)


def _sc_gather2(a, b, src, dst, d):
    @functools.partial(
        pl.kernel,
        mesh=_sc_mesh(),
        out_type=jax.ShapeDtypeStruct((E, d), jnp.float32),
        scratch_types=[
            pltpu.VMEM((EW,), jnp.int32),
            pltpu.VMEM((EW,), jnp.int32),
            pltpu.VMEM((CG, d), jnp.float32),
            pltpu.VMEM((CG, d), jnp.float32),
            pltpu.VMEM((CG, d), jnp.float32),
            pltpu.VMEM((CG, d), jnp.float32),
            pltpu.VMEM((CG, d), jnp.float32),
            pltpu.VMEM((CG, d), jnp.float32),
            pltpu.SemaphoreType.DMA,
            pltpu.SemaphoreType.DMA,
            pltpu.SemaphoreType.DMA,
            pltpu.SemaphoreType.DMA,
            pltpu.SemaphoreType.DMA,
            pltpu.SemaphoreType.DMA,
        ],
    )
    def k(a_hbm, b_hbm, src_hbm, dst_hbm, g_hbm, idx_s, idx_d,
          ba0, bb0, bo0, ba1, bb1, bo1, sa0, sb0, sw0, sa1, sb1, sw1):
        c = lax.axis_index("c")
        s = lax.axis_index("s")
        wid = s * NC + c
        slots = ((ba0, bb0, bo0, sa0, sb0, sw0),
                 (ba1, bb1, bo1, sa1, sb1, sw1))

        pltpu.sync_copy(src_hbm.at[pl.ds(wid * EW, EW)], idx_s)
        pltpu.sync_copy(dst_hbm.at[pl.ds(wid * EW, EW)], idx_d)

        def issue(j, t):
            ba, bb, _, sa, sb, _ = slots[t]
            pltpu.async_copy(a_hbm.at[idx_s.at[pl.ds(j * CG, CG)]], ba, sa)
            pltpu.async_copy(b_hbm.at[idx_d.at[pl.ds(j * CG, CG)]], bb, sb)

        def wait_g(t):
            ba, bb, _, sa, sb, _ = slots[t]
            pltpu.make_async_copy(a_hbm.at[pl.ds(0, CG)], ba, sa).wait()
            pltpu.make_async_copy(a_hbm.at[pl.ds(0, CG)], bb, sb).wait()

        def wait_w(t):
            _, _, bo, _, _, sw = slots[t]
            pltpu.make_async_copy(bo, g_hbm.at[pl.ds(0, CG)], sw).wait()

        def compute(t):
            ba, bb, bo, _, _, _ = slots[t]

            @plsc.parallel_loop(0, CG, unroll=4)
            def _(e):
                for q in range(d // LANES):
                    sl = pl.ds(q * LANES, LANES)
                    bo[e, sl] = ba[e, sl] + bb[e, sl]

        def wb(j, t):
            _, _, bo, _, _, sw = slots[t]
            pltpu.async_copy(bo, g_hbm.at[pl.ds(wid * EW + j * CG, CG)], sw)

        issue(0, 0)
        issue(1, 1)
        wait_g(0)
        compute(0)
        issue(2, 0)
        wb(0, 0)
        wait_g(1)
        compute(1)
        issue(3, 1)
        wb(1, 1)

        @pl.loop(1, (NCG - 1) // 2)
        def _pairs(k2):
            g = k2 * 2
            wait_g(0)
            wait_w(0)
            compute(0)
            issue(g + 2, 0)
            wb(g, 0)
            wait_g(1)
            wait_w(1)
            compute(1)

            @pl.when(g + 3 < NCG)
            def _():
                issue(g + 3, 1)

            wb(g + 1, 1)

        wait_g(0)
        wait_w(0)
        compute(0)
        wb(NCG - 1, 0)
        wait_w(0)
        wait_w(1)

    return k(a, b, src, dst)


def kernel(x, edge_index, edge_attr, params):
    p = params
    src = edge_index[0]
    dst = edge_index[1]
    zeros_n = jnp.zeros((N, HP), jnp.float32)

    h = _tc_node_init(x, _pad2(p['node_W'], F_IN, HP), _pad1(p['node_b'], HP))
    ea = _tc_edge_init(edge_attr, _pad2(p['edge_W'], E_DIM, HP),
                       _pad1(p['edge_b'], HP))

    for i in range(2):
        agg = _sc_msg_agg(h, ea, src, dst, zeros_n)
        w1 = p['emlp%d_W1' % i]
        h, a, b = _tc_node_update(
            h, agg,
            _pad2(p['conv%d_W1' % i], HP, HP), _pad1(p['conv%d_b1' % i], HP),
            _pad2(p['conv%d_W2' % i], HP, HP), _pad1(p['conv%d_b2' % i], HP),
            _pad1(p['bn%d_g' % i], HP), _pad1(p['bn%d_b' % i], HP),
            _pad2(w1[:H], HP, HP), _pad2(w1[H:2 * H], HP, HP))
        g = _sc_gather2(a, b, src, dst, HP)
        ea = _tc_edge_update(
            g, ea,
            _pad2(w1[2 * H:], HP, HP), _pad1(p['emlp%d_b1' % i], HP),
            _pad2(p['emlp%d_W2' % i], HP, HP), _pad1(p['emlp%d_b2' % i], HP))

    w1 = p['mlp_W1']
    pn, qn = _tc_pq(h, _pad2(w1[:H], HP, DP), _pad2(w1[H:2 * H], HP, DP))
    gf = _sc_gather2(pn, qn, src, dst, DP)
    out = _tc_final(
        gf, ea,
        _pad2(w1[2 * H:], HP, DP), _pad1(p['mlp_b1'], DP),
        _pad2(p['mlp_W2'], DP, 32), _pad1(p['mlp_b2'], 32),
        _pad2(p['mlp_W3'], 32, 8), _pad1(p['mlp_b3'], 8))
    return out[:, :2]
